# baseline (speedup 1.0000x reference)
import jax
import jax.numpy as jnp
from jax import lax
from jax.experimental import pallas as pl
from jax.experimental.pallas import tpu as pltpu

_B, _C, _H, _W, _E = 2, 384, 48, 48, 128
_N = _H * _W
_SIGMA = 0.05
_SCALE = float(_E) ** (-0.5)
_NEG = float(jnp.finfo(jnp.float32).min)
_LOG2E = 1.4426950408889634
_CB = 200.0 * _LOG2E / ((_H - 1) * (_H - 1))

_BLK_Q = 768
_NQ = _N // _BLK_Q


def _coords_i(idx_i32):
    gi = (idx_i32 // _W).astype(jnp.float32)
    gj = (idx_i32 % _W).astype(jnp.float32)
    return gi, gj


def _attn_kernel(x_blk_ref, x_full_ref, m_ref, wq_ref, wk_ref, wv_ref,
                 wp_ref, o_ref, k_s, v_s):
    qb = pl.program_id(1)

    @pl.when(qb == 0)
    def _project_kv():
        xf = x_full_ref[0].astype(jnp.bfloat16)
        k_s[...] = jax.lax.dot_general(
            xf, wk_ref[...].astype(jnp.bfloat16),
            (((0,), (1,)), ((), ())),
            preferred_element_type=jnp.float32,
        ).astype(jnp.bfloat16)
        v_s[...] = jax.lax.dot_general(
            xf, wv_ref[...].astype(jnp.bfloat16),
            (((0,), (1,)), ((), ())),
            preferred_element_type=jnp.float32,
        ).astype(jnp.bfloat16)

    q = (jax.lax.dot_general(
        wq_ref[...].astype(jnp.bfloat16), x_blk_ref[0].astype(jnp.bfloat16),
        (((1,), (0,)), ((), ())),
        preferred_element_type=jnp.float32,
    ) * (_SCALE * _LOG2E)).astype(jnp.bfloat16)

    qidx = qb * _BLK_Q + jax.lax.broadcasted_iota(jnp.int32, (1, _BLK_Q), 1)
    qgi, qgj = _coords_i(qidx)
    qc3 = -0.5 * (qgi * qgi + qgj * qgj)
    qi2 = jnp.concatenate([qgi, qgj, qc3], axis=0).astype(jnp.bfloat16)
    kidx = jax.lax.broadcasted_iota(jnp.int32, (1, _N), 1)
    kgi, kgj = _coords_i(kidx)
    ones = jnp.ones((1, _N), jnp.float32)
    ki2 = jnp.concatenate([kgi, kgj, ones], axis=0).astype(jnp.bfloat16)

    col = jnp.where(m_ref[0] > 0, _NEG,
                    (-_CB) * (kgi * kgi + kgj * kgj))

    s = jax.lax.dot_general(
        q, k_s[...], (((0,), (1,)), ((), ())),
        preferred_element_type=jnp.float32,
    )
    cross = jax.lax.dot_general(
        qi2, ki2, (((0,), (0,)), ((), ())),
        preferred_element_type=jnp.float32,
    )
    s = s + (cross * (2.0 * _CB) + col)
    p = jnp.exp2(s)
    l = jnp.sum(p, axis=1, keepdims=True)
    agg = jax.lax.dot_general(
        p.astype(jnp.bfloat16), v_s[...], (((1,), (0,)), ((), ())),
        preferred_element_type=jnp.float32,
    )
    agg = (agg * (1.0 / jnp.maximum(l, 1e-30))).astype(jnp.bfloat16)
    delta_t = jax.lax.dot_general(
        wp_ref[...].astype(jnp.bfloat16), agg, (((1,), (1,)), ((), ())),
        preferred_element_type=jnp.float32,
    )
    m_blk = m_ref[0, :, pl.ds(qb * _BLK_Q, _BLK_Q)]
    o_ref[0] = x_blk_ref[0] + jnp.where(m_blk > 0, delta_t, 0.0)


@jax.jit
def kernel(feat, mask_hi, Wq, Wk, Wv, Wp):
    x = feat.reshape(_B, _C, _N)
    m = mask_hi.reshape(_B, 1, _N)

    out = pl.pallas_call(
        _attn_kernel,
        grid=(_B, _NQ),
        in_specs=[
            pl.BlockSpec((1, _C, _BLK_Q), lambda b, q: (b, 0, q)),
            pl.BlockSpec((1, _C, _N), lambda b, q: (b, 0, 0)),
            pl.BlockSpec((1, 1, _N), lambda b, q: (b, 0, 0)),
            pl.BlockSpec((_E, _C), lambda b, q: (0, 0)),
            pl.BlockSpec((_E, _C), lambda b, q: (0, 0)),
            pl.BlockSpec((_E, _C), lambda b, q: (0, 0)),
            pl.BlockSpec((_C, _E), lambda b, q: (0, 0)),
        ],
        out_specs=pl.BlockSpec((1, _C, _BLK_Q), lambda b, q: (b, 0, q)),
        out_shape=jax.ShapeDtypeStruct((_B, _C, _N), jnp.float32),
        scratch_shapes=[
            pltpu.VMEM((_N, _E), jnp.bfloat16),
            pltpu.VMEM((_N, _E), jnp.bfloat16),
        ],
    )(x, x, m, Wq, Wk, Wv, Wp)

    return out.reshape(_B, _C, _H, _W)


# speedup vs baseline: 1.0146x; 1.0146x over previous
import jax
import jax.numpy as jnp
from jax import lax
from jax.experimental import pallas as pl
from jax.experimental.pallas import tpu as pltpu

_B, _C, _H, _W, _E = 2, 384, 48, 48, 128
_N = _H * _W
_SIGMA = 0.05
_SCALE = float(_E) ** (-0.5)
_NEG = float(jnp.finfo(jnp.float32).min)
_LOG2E = 1.4426950408889634
_CB = 200.0 * _LOG2E / ((_H - 1) * (_H - 1))

_BLK_Q = 1152
_NQ = _N // _BLK_Q


def _coords_i(idx_i32):
    gi = (idx_i32 // _W).astype(jnp.float32)
    gj = (idx_i32 % _W).astype(jnp.float32)
    return gi, gj


def _attn_kernel(x_blk_ref, x_full_ref, m_ref, wq_ref, wk_ref, wv_ref,
                 wp_ref, o_ref, k_s, v_s):
    qb = pl.program_id(1)

    @pl.when(qb == 0)
    def _project_kv():
        xf = x_full_ref[0].astype(jnp.bfloat16)
        k_s[...] = jax.lax.dot_general(
            xf, wk_ref[...].astype(jnp.bfloat16),
            (((0,), (1,)), ((), ())),
            preferred_element_type=jnp.float32,
        ).astype(jnp.bfloat16)
        v_s[...] = jax.lax.dot_general(
            xf, wv_ref[...].astype(jnp.bfloat16),
            (((0,), (1,)), ((), ())),
            preferred_element_type=jnp.float32,
        ).astype(jnp.bfloat16)

    q = (jax.lax.dot_general(
        wq_ref[...].astype(jnp.bfloat16), x_blk_ref[0].astype(jnp.bfloat16),
        (((1,), (0,)), ((), ())),
        preferred_element_type=jnp.float32,
    ) * (_SCALE * _LOG2E)).astype(jnp.bfloat16)

    qidx = qb * _BLK_Q + jax.lax.broadcasted_iota(jnp.int32, (1, _BLK_Q), 1)
    qgi, qgj = _coords_i(qidx)
    qc3 = -0.5 * (qgi * qgi + qgj * qgj)
    qi2 = jnp.concatenate([qgi, qgj, qc3], axis=0).astype(jnp.bfloat16)
    kidx = jax.lax.broadcasted_iota(jnp.int32, (1, _N), 1)
    kgi, kgj = _coords_i(kidx)
    ones = jnp.ones((1, _N), jnp.float32)
    ki2 = jnp.concatenate([kgi, kgj, ones], axis=0).astype(jnp.bfloat16)

    col = jnp.where(m_ref[0] > 0, _NEG,
                    (-_CB) * (kgi * kgi + kgj * kgj))

    s = jax.lax.dot_general(
        q, k_s[...], (((0,), (1,)), ((), ())),
        preferred_element_type=jnp.float32,
    )
    cross = jax.lax.dot_general(
        qi2, ki2, (((0,), (0,)), ((), ())),
        preferred_element_type=jnp.float32,
    )
    s = s + (cross * (2.0 * _CB) + col)
    p = jnp.exp2(s)
    l = jnp.sum(p, axis=1, keepdims=True)
    agg = jax.lax.dot_general(
        p.astype(jnp.bfloat16), v_s[...], (((1,), (0,)), ((), ())),
        preferred_element_type=jnp.float32,
    )
    agg = (agg * (1.0 / jnp.maximum(l, 1e-30))).astype(jnp.bfloat16)
    delta_t = jax.lax.dot_general(
        wp_ref[...].astype(jnp.bfloat16), agg, (((1,), (1,)), ((), ())),
        preferred_element_type=jnp.float32,
    )
    m_blk = m_ref[0, :, pl.ds(qb * _BLK_Q, _BLK_Q)]
    o_ref[0] = x_blk_ref[0] + jnp.where(m_blk > 0, delta_t, 0.0)


@jax.jit
def kernel(feat, mask_hi, Wq, Wk, Wv, Wp):
    x = feat.reshape(_B, _C, _N)
    m = mask_hi.reshape(_B, 1, _N)

    out = pl.pallas_call(
        _attn_kernel,
        grid=(_B, _NQ),
        in_specs=[
            pl.BlockSpec((1, _C, _BLK_Q), lambda b, q: (b, 0, q)),
            pl.BlockSpec((1, _C, _N), lambda b, q: (b, 0, 0)),
            pl.BlockSpec((1, 1, _N), lambda b, q: (b, 0, 0)),
            pl.BlockSpec((_E, _C), lambda b, q: (0, 0)),
            pl.BlockSpec((_E, _C), lambda b, q: (0, 0)),
            pl.BlockSpec((_E, _C), lambda b, q: (0, 0)),
            pl.BlockSpec((_C, _E), lambda b, q: (0, 0)),
        ],
        out_specs=pl.BlockSpec((1, _C, _BLK_Q), lambda b, q: (b, 0, q)),
        out_shape=jax.ShapeDtypeStruct((_B, _C, _N), jnp.float32),
        scratch_shapes=[
            pltpu.VMEM((_N, _E), jnp.bfloat16),
            pltpu.VMEM((_N, _E), jnp.bfloat16),
        ],
    )(x, x, m, Wq, Wk, Wv, Wp)

    return out.reshape(_B, _C, _H, _W)
